# Initial kernel scaffold; baseline (speedup 1.0000x reference)
import jax
import jax.numpy as jnp
from jax.experimental import pallas as pl


def kernel(x_input, W1, b1, W2, b2, batch, batch_num):
    raise NotImplementedError("write your pallas kernel here")


# speedup vs baseline: 5.7959x; 5.7959x over previous
import jax
import jax.numpy as jnp
from jax import lax
from jax.experimental import pallas as pl
from jax.experimental.pallas import tpu as pltpu

_N = 100000
_D = 256
_B = 64
_H = 16
_BLK = 4000
_STEPS = _N // _BLK


def _se_body(x_ref, ids_ref, w1t_ref, b1_ref, w2t_ref, b2_ref, out_ref,
             acc_ref, cnt_ref, attn_ref):
    p = pl.program_id(0)
    i = pl.program_id(1)
    ids = ids_ref[0, 0, :]

    @pl.when(p == 0)
    def _phase0():
        onehot_t = (ids[None, :] == lax.broadcasted_iota(
            jnp.int32, (_B, _BLK), 0)).astype(jnp.float32)
        partial = lax.dot_general(
            onehot_t, x_ref[...], (((1,), (0,)), ((), ())),
            preferred_element_type=jnp.float32,
            precision=lax.Precision.HIGHEST)
        pcnt = jnp.sum(onehot_t, axis=1, keepdims=True)

        @pl.when(i == 0)
        def _init():
            acc_ref[...] = partial
            cnt_ref[...] = pcnt

        @pl.when(i > 0)
        def _accum():
            acc_ref[...] += partial
            cnt_ref[...] += pcnt

    @pl.when(jnp.logical_and(p == 1, i == 0))
    def _mlp():
        avg = acc_ref[...] / jnp.maximum(cnt_ref[...], 1.0)
        h = jnp.maximum(
            lax.dot_general(avg, w1t_ref[...], (((1,), (0,)), ((), ())),
                            preferred_element_type=jnp.float32,
                            precision=lax.Precision.HIGHEST) + b1_ref[...],
            0.0)
        z = lax.dot_general(h, w2t_ref[...], (((1,), (0,)), ((), ())),
                            preferred_element_type=jnp.float32,
                            precision=lax.Precision.HIGHEST) + b2_ref[...]
        attn_ref[...] = jax.nn.sigmoid(z)

    @pl.when(p == 1)
    def _phase1():
        onehot = (ids[:, None] == lax.broadcasted_iota(
            jnp.int32, (_BLK, _B), 1)).astype(jnp.float32)
        scale = lax.dot_general(
            onehot, attn_ref[...], (((1,), (0,)), ((), ())),
            preferred_element_type=jnp.float32,
            precision=lax.Precision.HIGHEST)
        out_ref[...] = x_ref[...] * scale


def kernel(x_input, W1, b1, W2, b2, batch, batch_num):
    del batch_num
    ids3 = batch.astype(jnp.int32).reshape(_STEPS, 1, _BLK)
    w1t = W1.T
    w2t = W2.T
    b1r = b1.reshape(1, _H)
    b2r = b2.reshape(1, _D)

    return pl.pallas_call(
        _se_body,
        grid=(2, _STEPS),
        in_specs=[
            pl.BlockSpec((_BLK, _D), lambda p, i: (i, 0)),
            pl.BlockSpec((1, 1, _BLK), lambda p, i: (i, 0, 0)),
            pl.BlockSpec((_D, _H), lambda p, i: (0, 0)),
            pl.BlockSpec((1, _H), lambda p, i: (0, 0)),
            pl.BlockSpec((_H, _D), lambda p, i: (0, 0)),
            pl.BlockSpec((1, _D), lambda p, i: (0, 0)),
        ],
        out_specs=pl.BlockSpec((_BLK, _D), lambda p, i: (i * p, 0)),
        out_shape=jax.ShapeDtypeStruct((_N, _D), jnp.float32),
        scratch_shapes=[
            pltpu.VMEM((_B, _D), jnp.float32),
            pltpu.VMEM((_B, 1), jnp.float32),
            pltpu.VMEM((_B, _D), jnp.float32),
        ],
        compiler_params=pltpu.CompilerParams(
            dimension_semantics=("arbitrary", "arbitrary")),
    )(x_input, ids3, w1t, b1r, w2t, b2r)
